# baseline (speedup 1.0000x reference)
import functools

import jax
import jax.numpy as jnp
from jax import lax
from jax.experimental import pallas as pl
from jax.experimental.pallas import tpu as pltpu
from jax.experimental.pallas import tpu_sc as plsc

_VB = 1792
_NSLOT = 4
_NGATHER = 2


def _make_transpose(vocab: int, dim: int):
    info = plsc.get_sparse_core_info()
    nc, ns = info.num_cores, info.num_subcores
    nw = nc * ns
    n_stripes = vocab // _VB
    resid = vocab - n_stripes * _VB
    per_w = (n_stripes + nw - 1) // nw
    q_rows = vocab * dim // 128

    mesh = plsc.VectorSubcoreMesh(core_axis_name="c", subcore_axis_name="s")

    @functools.partial(
        pl.kernel,
        mesh=mesh,
        out_type=jax.ShapeDtypeStruct((q_rows, 128), jnp.float32),
        scratch_types=[
            pltpu.VMEM((dim, _VB), jnp.float32),
            pltpu.VMEM((_VB * dim // 128, 128), jnp.float32),
        ],
        compiler_params=pltpu.CompilerParams(
            use_tc_tiling_on_sc=True, needs_layout_passes=False
        ),
    )
    def transpose_kernel(tab_hbm, tail_hbm, q_hbm, t32, qb):
        wid = lax.axis_index("s") * nc + lax.axis_index("c")
        j16 = lax.iota(jnp.int32, 16)
        rbase = j16 >> 2
        lbase = (j16 & 3) * dim

        def do_stripe(v0, vb, qrow0, qrowsn):
            for tr in range(dim // 8):
                pltpu.sync_copy(
                    tab_hbm.at[pl.ds(8 * tr, 8), pl.ds(v0, vb)],
                    t32.at[pl.ds(8 * tr, 8), pl.ds(0, vb)],
                )

            def body(c, carry):
                ridx = c * 4 + rbase
                for f in range(dim):
                    x = t32[f, pl.ds(c * 16, 16)]
                    plsc.store_scatter(qb, [ridx, lbase + f], x)
                return carry

            lax.fori_loop(0, vb // 16, body, 0)
            pltpu.sync_copy(
                qb.at[pl.ds(0, qrowsn), :], q_hbm.at[pl.ds(qrow0, qrowsn), :]
            )

        for i in range(per_w):
            s_ = wid + i * nw

            @pl.when(s_ < n_stripes)
            def _():
                v0 = pl.multiple_of(s_ * _VB, 128)
                q0 = pl.multiple_of(s_ * (_VB * dim // 128), 8)
                do_stripe(v0, _VB, q0, _VB * dim // 128)

        if resid:
            tail_rows = resid * dim // 128

            @pl.when(wid == nw - 1)
            def _():
                pltpu.sync_copy(tail_hbm, qb.at[pl.ds(0, tail_rows), :])
                pltpu.sync_copy(
                    qb.at[pl.ds(0, tail_rows), :],
                    q_hbm.at[pl.ds(n_stripes * (_VB * dim // 128), tail_rows), :],
                )

    return transpose_kernel


def _make_gather(n_rows: int, dim: int, chunk: int):
    info = plsc.get_sparse_core_info()
    nc, ns = info.num_cores, info.num_subcores
    nw = nc * ns
    assert n_rows % (nw * chunk) == 0
    b_per_w = n_rows // nw
    n_iters = b_per_w // chunk

    mesh = plsc.VectorSubcoreMesh(core_axis_name="c", subcore_axis_name="s")

    @functools.partial(
        pl.kernel,
        mesh=mesh,
        out_type=jax.ShapeDtypeStruct((n_rows, dim), jnp.float32),
        scratch_types=[
            pltpu.VMEM((b_per_w,), jnp.int32),
            pltpu.VMEM((_NSLOT, chunk, dim), jnp.float32),
            pltpu.SemaphoreType.DMA((_NSLOT,)),
            pltpu.SemaphoreType.DMA((_NSLOT,)),
        ],
        compiler_params=pltpu.CompilerParams(use_tc_tiling_on_sc=False),
    )
    def gather_kernel(table_hbm, idx_hbm, out_hbm, idx_v, rows_v, gsem, osem):
        wid = lax.axis_index("s") * nc + lax.axis_index("c")
        base = wid * b_per_w

        pltpu.sync_copy(idx_hbm.at[pl.ds(base, b_per_w)], idx_v)

        def gather_copy(i):
            s = i % _NSLOT
            return pltpu.make_async_copy(
                table_hbm.at[idx_v.at[pl.ds(i * chunk, chunk)]],
                rows_v.at[s],
                gsem.at[s],
            )

        def out_copy(i):
            s = i % _NSLOT
            return pltpu.make_async_copy(
                rows_v.at[s],
                out_hbm.at[pl.ds(base + i * chunk, chunk)],
                osem.at[s],
            )

        for i in range(min(_NGATHER, n_iters)):
            gather_copy(i).start()
        outs_pending = []
        for i in range(n_iters):
            gather_copy(i).wait()
            out_copy(i).start()
            outs_pending.append(i)
            nxt = i + _NGATHER
            if nxt < n_iters:
                reuse = nxt - _NSLOT
                if reuse >= 0:
                    out_copy(reuse).wait()
                    outs_pending.remove(reuse)
                gather_copy(nxt).start()
        for i in outs_pending:
            out_copy(i).wait()

    return gather_kernel


@jax.jit
def kernel(indices, feat_table):
    batch, hist = indices.shape
    vocab, dim = feat_table.shape
    n_rows = batch * hist
    n_stripes = vocab // _VB
    v_main = n_stripes * _VB
    tail = feat_table[v_main:, :].reshape((vocab - v_main) * dim // 128, 128)
    q = _make_transpose(vocab, dim)(feat_table.T, tail)
    table_lin = q.reshape(vocab, dim)
    idx_flat = indices.reshape(n_rows).astype(jnp.int32)
    out = _make_gather(n_rows, dim, chunk=800)(table_lin, idx_flat)
    return out.reshape(batch, hist, dim)


# speedup vs baseline: 1.1984x; 1.1984x over previous
import functools

import jax
import jax.numpy as jnp
from jax import lax
from jax.experimental import pallas as pl
from jax.experimental.pallas import tpu as pltpu
from jax.experimental.pallas import tpu_sc as plsc

_VB = 1792
_NSLOT = 4
_NGATHER = 2


def _make_transpose(vocab: int, dim: int):
    info = plsc.get_sparse_core_info()
    nc, ns = info.num_cores, info.num_subcores
    nw = nc * ns
    n_stripes = vocab // _VB
    resid = vocab - n_stripes * _VB
    per_w = (n_stripes + nw - 1) // nw
    q_rows = vocab * dim // 128

    mesh = plsc.VectorSubcoreMesh(core_axis_name="c", subcore_axis_name="s")

    @functools.partial(
        pl.kernel,
        mesh=mesh,
        out_type=jax.ShapeDtypeStruct((q_rows, 128), jnp.float32),
        scratch_types=[
            pltpu.VMEM((dim, _VB), jnp.float32),
            pltpu.VMEM((_VB * dim // 128, 128), jnp.float32),
        ],
        compiler_params=pltpu.CompilerParams(
            use_tc_tiling_on_sc=True, needs_layout_passes=False
        ),
    )
    def transpose_kernel(tab_hbm, tail_hbm, q_hbm, t32, qb):
        wid = lax.axis_index("s") * nc + lax.axis_index("c")
        j16 = lax.iota(jnp.int32, 16)
        rbase = j16 >> 2
        lbase = (j16 & 3) * dim

        def do_stripe(v0, vb, qrow0, qrowsn):
            for tr in range(dim // 8):
                pltpu.sync_copy(
                    tab_hbm.at[pl.ds(8 * tr, 8), pl.ds(v0, vb)],
                    t32.at[pl.ds(8 * tr, 8), pl.ds(0, vb)],
                )

            @plsc.parallel_loop(0, vb // 16, unroll=1)
            def body(c):
                ridx = c * 4 + rbase
                for f in range(dim):
                    x = t32[f, pl.ds(c * 16, 16)]
                    plsc.store_scatter(qb, [ridx, lbase + f], x)
            pltpu.sync_copy(
                qb.at[pl.ds(0, qrowsn), :], q_hbm.at[pl.ds(qrow0, qrowsn), :]
            )

        for i in range(per_w):
            s_ = wid + i * nw

            @pl.when(s_ < n_stripes)
            def _():
                v0 = pl.multiple_of(s_ * _VB, 128)
                q0 = pl.multiple_of(s_ * (_VB * dim // 128), 8)
                do_stripe(v0, _VB, q0, _VB * dim // 128)

        if resid:
            tail_rows = resid * dim // 128

            @pl.when(wid == nw - 1)
            def _():
                pltpu.sync_copy(tail_hbm, qb.at[pl.ds(0, tail_rows), :])
                pltpu.sync_copy(
                    qb.at[pl.ds(0, tail_rows), :],
                    q_hbm.at[pl.ds(n_stripes * (_VB * dim // 128), tail_rows), :],
                )

    return transpose_kernel


def _make_gather(n_rows: int, dim: int, chunk: int):
    info = plsc.get_sparse_core_info()
    nc, ns = info.num_cores, info.num_subcores
    nw = nc * ns
    assert n_rows % (nw * chunk) == 0
    b_per_w = n_rows // nw
    n_iters = b_per_w // chunk

    mesh = plsc.VectorSubcoreMesh(core_axis_name="c", subcore_axis_name="s")

    @functools.partial(
        pl.kernel,
        mesh=mesh,
        out_type=jax.ShapeDtypeStruct((n_rows, dim), jnp.float32),
        scratch_types=[
            pltpu.VMEM((b_per_w,), jnp.int32),
            pltpu.VMEM((_NSLOT, chunk, dim), jnp.float32),
            pltpu.SemaphoreType.DMA((_NSLOT,)),
            pltpu.SemaphoreType.DMA((_NSLOT,)),
        ],
        compiler_params=pltpu.CompilerParams(use_tc_tiling_on_sc=False),
    )
    def gather_kernel(table_hbm, idx_hbm, out_hbm, idx_v, rows_v, gsem, osem):
        wid = lax.axis_index("s") * nc + lax.axis_index("c")
        base = wid * b_per_w

        pltpu.sync_copy(idx_hbm.at[pl.ds(base, b_per_w)], idx_v)

        def gather_copy(i):
            s = i % _NSLOT
            return pltpu.make_async_copy(
                table_hbm.at[idx_v.at[pl.ds(i * chunk, chunk)]],
                rows_v.at[s],
                gsem.at[s],
            )

        def out_copy(i):
            s = i % _NSLOT
            return pltpu.make_async_copy(
                rows_v.at[s],
                out_hbm.at[pl.ds(base + i * chunk, chunk)],
                osem.at[s],
            )

        for i in range(min(_NGATHER, n_iters)):
            gather_copy(i).start()
        outs_pending = []
        for i in range(n_iters):
            gather_copy(i).wait()
            out_copy(i).start()
            outs_pending.append(i)
            nxt = i + _NGATHER
            if nxt < n_iters:
                reuse = nxt - _NSLOT
                if reuse >= 0:
                    out_copy(reuse).wait()
                    outs_pending.remove(reuse)
                gather_copy(nxt).start()
        for i in outs_pending:
            out_copy(i).wait()

    return gather_kernel


@jax.jit
def kernel(indices, feat_table):
    batch, hist = indices.shape
    vocab, dim = feat_table.shape
    n_rows = batch * hist
    n_stripes = vocab // _VB
    v_main = n_stripes * _VB
    tail = feat_table[v_main:, :].reshape((vocab - v_main) * dim // 128, 128)
    q = _make_transpose(vocab, dim)(feat_table.T, tail)
    table_lin = q.reshape(vocab, dim)
    idx_flat = indices.reshape(n_rows).astype(jnp.int32)
    out = _make_gather(n_rows, dim, chunk=800)(table_lin, idx_flat)
    return out.reshape(batch, hist, dim)


# speedup vs baseline: 1.3537x; 1.1296x over previous
import functools

import jax
import jax.numpy as jnp
from jax import lax
from jax.experimental import pallas as pl
from jax.experimental.pallas import tpu as pltpu
from jax.experimental.pallas import tpu_sc as plsc

_NSLOT = 4
_NGATHER = 2


def _make_gather(n_rows: int, dim: int, chunk: int):
    info = plsc.get_sparse_core_info()
    nc, ns = info.num_cores, info.num_subcores
    nw = nc * ns
    assert n_rows % (nw * chunk) == 0
    b_per_w = n_rows // nw
    n_iters = b_per_w // chunk

    mesh = plsc.VectorSubcoreMesh(core_axis_name="c", subcore_axis_name="s")

    @functools.partial(
        pl.kernel,
        mesh=mesh,
        out_type=jax.ShapeDtypeStruct((n_rows, dim), jnp.float32),
        scratch_types=[
            pltpu.VMEM((b_per_w,), jnp.int32),
            pltpu.VMEM((_NSLOT, chunk, dim), jnp.float32),
            pltpu.SemaphoreType.DMA((_NSLOT,)),
            pltpu.SemaphoreType.DMA((_NSLOT,)),
        ],
        compiler_params=pltpu.CompilerParams(use_tc_tiling_on_sc=False),
    )
    def gather_kernel(table_hbm, idx_hbm, out_hbm, idx_v, rows_v, gsem, osem):
        wid = lax.axis_index("s") * nc + lax.axis_index("c")
        base = wid * b_per_w

        pltpu.sync_copy(idx_hbm.at[pl.ds(base, b_per_w)], idx_v)

        def gather_copy(i):
            s = i % _NSLOT
            return pltpu.make_async_copy(
                table_hbm.at[idx_v.at[pl.ds(i * chunk, chunk)]],
                rows_v.at[s],
                gsem.at[s],
            )

        def out_copy(i):
            s = i % _NSLOT
            return pltpu.make_async_copy(
                rows_v.at[s],
                out_hbm.at[pl.ds(base + i * chunk, chunk)],
                osem.at[s],
            )

        for i in range(min(_NGATHER, n_iters)):
            gather_copy(i).start()
        outs_pending = []
        for i in range(n_iters):
            gather_copy(i).wait()
            out_copy(i).start()
            outs_pending.append(i)
            nxt = i + _NGATHER
            if nxt < n_iters:
                reuse = nxt - _NSLOT
                if reuse >= 0:
                    out_copy(reuse).wait()
                    outs_pending.remove(reuse)
                gather_copy(nxt).start()
        for i in outs_pending:
            out_copy(i).wait()

    return gather_kernel


@jax.jit
def kernel(indices, feat_table):
    batch, hist = indices.shape
    vocab, dim = feat_table.shape
    n_rows = batch * hist
    idx_flat = indices.reshape(n_rows).astype(jnp.int32)
    out = _make_gather(n_rows, dim, chunk=800)(feat_table, idx_flat)
    return out.reshape(batch, hist, dim)
